# Initial kernel scaffold; baseline (speedup 1.0000x reference)
import jax
import jax.numpy as jnp
from jax.experimental import pallas as pl


def kernel(x1, edge_index1, e1, u1, batch1, x2, edge_index2, e2, u2, batch2, params):
    raise NotImplementedError("write your pallas kernel here")


# speedup vs baseline: 5.2395x; 5.2395x over previous
import functools

import jax
import jax.numpy as jnp
from jax import lax
from jax.experimental import pallas as pl
from jax.experimental.pallas import tpu as pltpu
import jax.experimental.pallas.tpu_sc as plsc

NC = 2
NS = 16
CHUNK = 80
HW = 128


def _relu(v):
    return jnp.maximum(v, 0.0)


def _sigmoid(v):
    return 1.0 / (1.0 + jnp.exp(-v))


def _ewb_body(e_ref, w_ref, b_ref, out_ref):
    out_ref[0] = jnp.dot(e_ref[0], w_ref[...],
                         preferred_element_type=jnp.float32) + b_ref[...]


def _ewb(e_s, w0e, b0, eb):
    g, e_tot, fe = e_s.shape
    grid = (g, e_tot // eb)
    return pl.pallas_call(
        _ewb_body,
        grid=grid,
        in_specs=[
            pl.BlockSpec((1, eb, fe), lambda i, j: (i, j, 0)),
            pl.BlockSpec((fe, 64), lambda i, j: (0, 0)),
            pl.BlockSpec((1, 64), lambda i, j: (0, 0)),
        ],
        out_specs=pl.BlockSpec((1, eb, 64), lambda i, j: (i, j, 0)),
        out_shape=jax.ShapeDtypeStruct((g, e_tot, 64), jnp.float32),
    )(e_s, w0e, b0)


def _pq_body(x_ref, u_ref, b_ref, wx_ref, wu_ref, pq_ref):
    nb = x_ref.shape[1]
    xw = jnp.dot(x_ref[0], wx_ref[...], preferred_element_type=jnp.float32)
    uw = jnp.dot(u_ref[0], wu_ref[...], preferred_element_type=jnp.float32)
    bvec = b_ref[0, 0, 0]
    oh = (bvec.reshape(nb, 1) ==
          lax.broadcasted_iota(jnp.int32, (nb, 16), 1)).astype(jnp.float32)
    ug = jnp.dot(oh, uw, preferred_element_type=jnp.float32)
    pq_ref[0] = jnp.concatenate([xw, ug - xw], axis=1)


def _pq(x_s, u_s, batch_r, w0x, w0u, nb):
    g, n_tot, fx = x_s.shape
    grid = (g, n_tot // nb)
    return pl.pallas_call(
        _pq_body,
        grid=grid,
        in_specs=[
            pl.BlockSpec((1, nb, fx), lambda i, j: (i, j, 0)),
            pl.BlockSpec((1, 16, fx), lambda i, j: (i, 0, 0)),
            pl.BlockSpec((1, 1, 1, nb), lambda i, j: (i, j, 0, 0)),
            pl.BlockSpec((fx, 64), lambda i, j: (0, 0)),
            pl.BlockSpec((fx, 64), lambda i, j: (0, 0)),
        ],
        out_specs=pl.BlockSpec((1, nb, 128), lambda i, j: (i, j, 0)),
        out_shape=jax.ShapeDtypeStruct((g, n_tot, 128), jnp.float32),
    )(x_s, u_s, batch_r, w0x, w0u)


def _edge_sc_body(n_pad, n_edges,
                  pq_hbm, ewb_hbm, src_hbm, dst_hbm, z_hbm, out_hbm,
                  idx_s, idx_d, buf_d, buf_s, buf_e, buf_h, s_sh,
                  sem_p, sem_q, sem_e):
    c = lax.axis_index("c")
    s = lax.axis_index("s")
    w = s * NC + c
    npart = n_pad // NS
    per_tile = n_edges // (NC * NS)
    nchunk = per_tile // CHUNK
    base0 = w * per_tile

    pltpu.sync_copy(z_hbm.at[pl.ds(s * npart, npart)],
                    s_sh.at[pl.ds(s * npart, npart)])

    def ones_row(r, carry):
        buf_h[r, pl.ds(64, 16)] = jnp.full((16,), 1.0, jnp.float32)
        for k in range(5, 8):
            buf_h[r, pl.ds(k * 16, 16)] = jnp.zeros((16,), jnp.float32)
        return carry
    lax.fori_loop(0, CHUNK, ones_row, 0)
    plsc.subcore_barrier()

    def chunk_body(j, carry):
        base = base0 + j * CHUNK
        pltpu.sync_copy(src_hbm.at[pl.ds(base, CHUNK)], idx_s.at[0])
        pltpu.sync_copy(dst_hbm.at[pl.ds(base, CHUNK)], idx_d.at[0])
        cp_p = pltpu.async_copy(pq_hbm.at[idx_d.at[0]], buf_d, sem_p)
        cp_q = pltpu.async_copy(pq_hbm.at[idx_s.at[0]], buf_s, sem_q)
        cp_e = pltpu.async_copy(ewb_hbm.at[pl.ds(base, CHUNK)], buf_e, sem_e)
        cp_p.wait()
        cp_q.wait()
        cp_e.wait()

        def row(r, inner):
            for k in range(4):
                sl = pl.ds(k * 16, 16)
                buf_h[r, sl] = _relu(buf_d[r, sl]
                                     + buf_s[r, pl.ds(64 + k * 16, 16)]
                                     + buf_e[r, sl])
            return inner
        lax.fori_loop(0, CHUNK, row, 0)
        pltpu.sync_copy(buf_h, s_sh.at[idx_d.at[0]], add=True)
        return carry
    lax.fori_loop(0, nchunk, chunk_body, 0)

    plsc.subcore_barrier()
    pltpu.sync_copy(s_sh.at[pl.ds(s * npart, npart)],
                    out_hbm.at[c, pl.ds(s * npart, npart)])


def _edge_scatter(pq, ewb, src, dst, zeros):
    n_pad = zeros.shape[0]
    n_edges = src.shape[0]
    mesh = plsc.VectorSubcoreMesh(core_axis_name="c", subcore_axis_name="s",
                                  num_cores=NC, num_subcores=NS)
    kern = pl.kernel(
        functools.partial(_edge_sc_body, n_pad, n_edges),
        out_type=jax.ShapeDtypeStruct((NC, n_pad, HW), jnp.float32),
        mesh=mesh,
        scratch_types=[
            pltpu.VMEM((1, CHUNK), jnp.int32),
            pltpu.VMEM((1, CHUNK), jnp.int32),
            pltpu.VMEM((CHUNK, 128), jnp.float32),
            pltpu.VMEM((CHUNK, 128), jnp.float32),
            pltpu.VMEM((CHUNK, 64), jnp.float32),
            pltpu.VMEM((CHUNK, HW), jnp.float32),
            pltpu.VMEM_SHARED((n_pad, HW), jnp.float32),
            pltpu.SemaphoreType.DMA,
            pltpu.SemaphoreType.DMA,
            pltpu.SemaphoreType.DMA,
        ],
    )
    return kern(pq, ewb, src, dst, zeros)


def _node_body(x_ref, u_ref, b_ref, s_ref,
               w1e_ref, wn0x_ref, wn0a_ref, wn0u_ref, bn0_ref,
               wn1_ref, bn1_ref,
               wa0x_ref, wa0u_ref, ba0_ref, wa1_ref, ba1_ref,
               wv0x_ref, wv0u_ref, bv0_ref, wv1_ref, bv1_ref,
               b1_ref, out_ref):
    i = pl.program_id(1)
    nb = x_ref.shape[1]
    dot = functools.partial(jnp.dot, preferred_element_type=jnp.float32)

    ssum = s_ref[0, 0] + s_ref[0, 1]
    s64 = ssum[:, :64]
    deg = ssum[:, 64:65]
    x = x_ref[0]
    uu = u_ref[0]
    bvec = b_ref[0, 0, 0]
    oh = (bvec.reshape(nb, 1) ==
          lax.broadcasted_iota(jnp.int32, (nb, 16), 1)).astype(jnp.float32)

    wf = dot(w1e_ref[...], wn0a_ref[...])
    bf = dot(b1_ref[...], wn0a_ref[...])

    z = _relu(dot(x, wn0x_ref[...]) + dot(s64, wf) + deg * bf
              + dot(oh, dot(uu, wn0u_ref[...])) + bn0_ref[...])
    x_h = dot(z, wn1_ref[...]) + bn1_ref[...]

    ga = _relu(dot(x_h, wa0x_ref[...]) + dot(oh, dot(uu, wa0u_ref[...]))
               + ba0_ref[...])
    attn = _sigmoid(dot(ga, wa1_ref[...]) + ba1_ref[...])
    gv = _relu(dot(x_h, wv0x_ref[...]) + dot(oh, dot(uu, wv0u_ref[...]))
               + bv0_ref[...])
    val = dot(gv, wv1_ref[...]) + bv1_ref[...]
    y = attn * val

    part = lax.dot_general(oh, y, (((0,), (0,)), ((), ())),
                           preferred_element_type=jnp.float32)

    @pl.when(i == 0)
    def _():
        out_ref[0] = part

    @pl.when(i > 0)
    def _():
        out_ref[0] += part


def _node_stage(x_s, u_s, batch_r, s_s, weights, nb):
    g, n_tot, fx = x_s.shape
    grid = (g, n_tot // nb)
    full = lambda shp: pl.BlockSpec(shp, lambda i, j: tuple(0 for _ in shp))
    in_specs = [
        pl.BlockSpec((1, nb, fx), lambda i, j: (i, j, 0)),
        pl.BlockSpec((1, 16, fx), lambda i, j: (i, 0, 0)),
        pl.BlockSpec((1, 1, 1, nb), lambda i, j: (i, j, 0, 0)),
        pl.BlockSpec((1, NC, nb, HW), lambda i, j: (i, 0, j, 0)),
    ] + [full(w.shape) for w in weights]
    return pl.pallas_call(
        _node_body,
        grid=grid,
        in_specs=in_specs,
        out_specs=pl.BlockSpec((1, 16, 64), lambda i, j: (i, 0, 0)),
        out_shape=jax.ShapeDtypeStruct((g, 16, 64), jnp.float32),
    )(x_s, u_s, batch_r, s_s, *weights)


def _final_body(u_ref, w0_ref, b0_ref, w1_ref, b1_ref, out_ref):
    dot = functools.partial(jnp.dot, preferred_element_type=jnp.float32)
    uh = jnp.concatenate([u_ref[0], u_ref[1]], axis=1)
    h = _relu(dot(uh, w0_ref[...]) + b0_ref[...])
    out_ref[...] = dot(h, w1_ref[...]) + b1_ref[...]


def _final(u_h, w0, b0, w1, b1):
    return pl.pallas_call(
        _final_body,
        out_shape=jax.ShapeDtypeStruct((16, w1.shape[1]), jnp.float32),
    )(u_h, w0, b0, w1, b1)


def kernel(x1, edge_index1, e1, u1, batch1, x2, edge_index2, e2, u2, batch2,
           params):
    n = x1.shape[0]
    n_edges = e1.shape[0]
    fx = x1.shape[1]
    fe = e1.shape[1]
    nb = 2000
    eb = 8000

    (w0, b0), (w1, b1) = params['edge']
    (wn0, bn0), (wn1, bn1) = params['node']
    (wa0, ba0), (wa1, ba1) = params['glob_a']
    (wv0, bv0), (wv1, bv1) = params['glob_v']
    (wf0, bf0), (wf1, bf1) = params['final']

    w0x = w0[:fx]
    w0e = w0[fx:fx + fe]
    w0u = w0[fx + fe:]
    wn0x = wn0[:fx]
    wn0a = wn0[fx:fx + 64]
    wn0u = wn0[fx + 64:]
    wa0x, wa0u = wa0[:64], wa0[64:]
    wv0x, wv0u = wv0[:64], wv0[64:]
    row = lambda v: v.reshape(1, -1)

    x_s = jnp.stack([x1, x2])
    e_s = jnp.stack([e1, e2])
    u_s = jnp.stack([u1, u2])
    batch_r = jnp.stack([batch1, batch2]).reshape(2, n // nb, 1, nb)

    ewb = _ewb(e_s, w0e, row(b0), eb)
    pq_s = _pq(x_s, u_s, batch_r, w0x, w0u, nb)

    n_pad = (-(-(n // NS) // 8) * 8) * NS
    zeros = jnp.zeros((n_pad, HW), jnp.float32)
    s1 = _edge_scatter(pq_s[0], ewb[0],
                       edge_index1[0], edge_index1[1], zeros)
    s2 = _edge_scatter(pq_s[1], ewb[1],
                       edge_index2[0], edge_index2[1], zeros)
    s_s = jnp.stack([s1, s2])

    weights = [w1, wn0x, wn0a, wn0u, row(bn0), wn1, row(bn1),
               wa0x, wa0u, row(ba0), wa1, row(ba1),
               wv0x, wv0u, row(bv0), wv1, row(bv1), row(b1)]
    u_h = _node_stage(x_s, u_s, batch_r, s_s, weights, nb)

    return _final(u_h, wf0, row(bf0), wf1, row(bf1))


# speedup vs baseline: 5.9777x; 1.1409x over previous
import functools

import jax
import jax.numpy as jnp
from jax import lax
from jax.experimental import pallas as pl
from jax.experimental.pallas import tpu as pltpu
import jax.experimental.pallas.tpu_sc as plsc

NC = 2
NS = 16
CHUNK = 40
HW = 128


def _relu(v):
    return jnp.maximum(v, 0.0)


def _sigmoid(v):
    return 1.0 / (1.0 + jnp.exp(-v))


def _ewb_body(e_ref, w_ref, b_ref, out_ref):
    out_ref[0] = jnp.dot(e_ref[0], w_ref[...],
                         preferred_element_type=jnp.float32) + b_ref[...]


def _ewb(e_s, w0e, b0, eb):
    g, e_tot, fe = e_s.shape
    grid = (g, e_tot // eb)
    return pl.pallas_call(
        _ewb_body,
        grid=grid,
        in_specs=[
            pl.BlockSpec((1, eb, fe), lambda i, j: (i, j, 0)),
            pl.BlockSpec((fe, 64), lambda i, j: (0, 0)),
            pl.BlockSpec((1, 64), lambda i, j: (0, 0)),
        ],
        out_specs=pl.BlockSpec((1, eb, 64), lambda i, j: (i, j, 0)),
        out_shape=jax.ShapeDtypeStruct((g, e_tot, 64), jnp.float32),
    )(e_s, w0e, b0)


def _pq_body(x_ref, u_ref, b_ref, wx_ref, wu_ref, pq_ref):
    nb = x_ref.shape[1]
    xw = jnp.dot(x_ref[0], wx_ref[...], preferred_element_type=jnp.float32)
    uw = jnp.dot(u_ref[0], wu_ref[...], preferred_element_type=jnp.float32)
    bvec = b_ref[0, 0, 0]
    oh = (bvec.reshape(nb, 1) ==
          lax.broadcasted_iota(jnp.int32, (nb, 16), 1)).astype(jnp.float32)
    ug = jnp.dot(oh, uw, preferred_element_type=jnp.float32)
    pq_ref[0] = jnp.concatenate([xw, ug - xw], axis=1)


def _pq(x_s, u_s, batch_r, w0x, w0u, nb):
    g, n_tot, fx = x_s.shape
    grid = (g, n_tot // nb)
    return pl.pallas_call(
        _pq_body,
        grid=grid,
        in_specs=[
            pl.BlockSpec((1, nb, fx), lambda i, j: (i, j, 0)),
            pl.BlockSpec((1, 16, fx), lambda i, j: (i, 0, 0)),
            pl.BlockSpec((1, 1, 1, nb), lambda i, j: (i, j, 0, 0)),
            pl.BlockSpec((fx, 64), lambda i, j: (0, 0)),
            pl.BlockSpec((fx, 64), lambda i, j: (0, 0)),
        ],
        out_specs=pl.BlockSpec((1, nb, 128), lambda i, j: (i, j, 0)),
        out_shape=jax.ShapeDtypeStruct((g, n_tot, 128), jnp.float32),
    )(x_s, u_s, batch_r, w0x, w0u)


def _edge_sc_body(n_pad, n_edges,
                  pq_hbm, ewb_hbm, src_hbm, dst_hbm, z_hbm, out_hbm,
                  idx_s, idx_d, buf_d, buf_s, buf_e, buf_h, s_sh,
                  sem_p0, sem_q0, sem_e0, sem_p1, sem_q1, sem_e1):
    c = lax.axis_index("c")
    s = lax.axis_index("s")
    w = s * NC + c
    npart = n_pad // NS
    per_tile = n_edges // (NC * NS)
    nchunk = per_tile // CHUNK
    base0 = w * per_tile
    sems = ((sem_p0, sem_q0, sem_e0), (sem_p1, sem_q1, sem_e1))

    pltpu.sync_copy(z_hbm.at[pl.ds(s * npart, npart)],
                    s_sh.at[pl.ds(s * npart, npart)])

    def ones_row(r, carry):
        buf_h[r, pl.ds(64, 16)] = jnp.full((16,), 1.0, jnp.float32)
        for k in range(5, 8):
            buf_h[r, pl.ds(k * 16, 16)] = jnp.zeros((16,), jnp.float32)
        return carry
    lax.fori_loop(0, CHUNK, ones_row, 0)
    plsc.subcore_barrier()

    def fire(b, base):
        sp, sq, se = sems[b]
        pltpu.sync_copy(src_hbm.at[pl.ds(base, CHUNK)], idx_s.at[b])
        pltpu.sync_copy(dst_hbm.at[pl.ds(base, CHUNK)], idx_d.at[b])
        pltpu.async_copy(pq_hbm.at[idx_d.at[b]], buf_d.at[b], sp)
        pltpu.async_copy(pq_hbm.at[idx_s.at[b]], buf_s.at[b], sq)
        pltpu.async_copy(ewb_hbm.at[pl.ds(base, CHUNK)], buf_e.at[b], se)

    def consume(b, base):
        sp, sq, se = sems[b]
        pltpu.make_async_copy(pq_hbm.at[idx_d.at[b]], buf_d.at[b], sp).wait()
        pltpu.make_async_copy(pq_hbm.at[idx_s.at[b]], buf_s.at[b], sq).wait()
        pltpu.make_async_copy(ewb_hbm.at[pl.ds(base, CHUNK)], buf_e.at[b],
                              se).wait()
        bd = buf_d.at[b]
        bs = buf_s.at[b]
        be = buf_e.at[b]

        def row(r, inner):
            for k in range(4):
                sl = pl.ds(k * 16, 16)
                buf_h[r, sl] = _relu(bd[r, sl]
                                     + bs[r, pl.ds(64 + k * 16, 16)]
                                     + be[r, sl])
            return inner
        lax.fori_loop(0, CHUNK, row, 0)
        pltpu.sync_copy(buf_h, s_sh.at[idx_d.at[b]], add=True)

    fire(0, base0)
    fire(1, base0 + CHUNK)

    def pair_body(j2, carry):
        j = j2 * 2
        for b in range(2):
            base = base0 + (j + b) * CHUNK
            consume(b, base)
            fire(b, base + 2 * CHUNK)
        return carry
    lax.fori_loop(0, (nchunk - 2) // 2, pair_body, 0)

    consume(0, base0 + (nchunk - 2) * CHUNK)
    consume(1, base0 + (nchunk - 1) * CHUNK)

    plsc.subcore_barrier()
    pltpu.sync_copy(s_sh.at[pl.ds(s * npart, npart)],
                    out_hbm.at[c, pl.ds(s * npart, npart)])


def _edge_scatter(pq, ewb, src, dst, zeros):
    n_pad = zeros.shape[0]
    n_edges = src.shape[0]
    mesh = plsc.VectorSubcoreMesh(core_axis_name="c", subcore_axis_name="s",
                                  num_cores=NC, num_subcores=NS)
    kern = pl.kernel(
        functools.partial(_edge_sc_body, n_pad, n_edges),
        out_type=jax.ShapeDtypeStruct((NC, n_pad, HW), jnp.float32),
        mesh=mesh,
        scratch_types=[
            pltpu.VMEM((2, CHUNK), jnp.int32),
            pltpu.VMEM((2, CHUNK), jnp.int32),
            pltpu.VMEM((2, CHUNK, 128), jnp.float32),
            pltpu.VMEM((2, CHUNK, 128), jnp.float32),
            pltpu.VMEM((2, CHUNK, 64), jnp.float32),
            pltpu.VMEM((CHUNK, HW), jnp.float32),
            pltpu.VMEM_SHARED((n_pad, HW), jnp.float32),
        ] + [pltpu.SemaphoreType.DMA] * 6,
    )
    return kern(pq, ewb, src, dst, zeros)


def _node_body(x_ref, u_ref, b_ref, s_ref,
               w1e_ref, wn0x_ref, wn0a_ref, wn0u_ref, bn0_ref,
               wn1_ref, bn1_ref,
               wa0x_ref, wa0u_ref, ba0_ref, wa1_ref, ba1_ref,
               wv0x_ref, wv0u_ref, bv0_ref, wv1_ref, bv1_ref,
               b1_ref, out_ref):
    i = pl.program_id(1)
    nb = x_ref.shape[1]
    dot = functools.partial(jnp.dot, preferred_element_type=jnp.float32)

    ssum = s_ref[0, 0] + s_ref[0, 1]
    s64 = ssum[:, :64]
    deg = ssum[:, 64:65]
    x = x_ref[0]
    uu = u_ref[0]
    bvec = b_ref[0, 0, 0]
    oh = (bvec.reshape(nb, 1) ==
          lax.broadcasted_iota(jnp.int32, (nb, 16), 1)).astype(jnp.float32)

    wf = dot(w1e_ref[...], wn0a_ref[...])
    bf = dot(b1_ref[...], wn0a_ref[...])

    z = _relu(dot(x, wn0x_ref[...]) + dot(s64, wf) + deg * bf
              + dot(oh, dot(uu, wn0u_ref[...])) + bn0_ref[...])
    x_h = dot(z, wn1_ref[...]) + bn1_ref[...]

    ga = _relu(dot(x_h, wa0x_ref[...]) + dot(oh, dot(uu, wa0u_ref[...]))
               + ba0_ref[...])
    attn = _sigmoid(dot(ga, wa1_ref[...]) + ba1_ref[...])
    gv = _relu(dot(x_h, wv0x_ref[...]) + dot(oh, dot(uu, wv0u_ref[...]))
               + bv0_ref[...])
    val = dot(gv, wv1_ref[...]) + bv1_ref[...]
    y = attn * val

    part = lax.dot_general(oh, y, (((0,), (0,)), ((), ())),
                           preferred_element_type=jnp.float32)

    @pl.when(i == 0)
    def _():
        out_ref[0] = part

    @pl.when(i > 0)
    def _():
        out_ref[0] += part


def _node_stage(x_s, u_s, batch_r, s_s, weights, nb):
    g, n_tot, fx = x_s.shape
    grid = (g, n_tot // nb)
    full = lambda shp: pl.BlockSpec(shp, lambda i, j: tuple(0 for _ in shp))
    in_specs = [
        pl.BlockSpec((1, nb, fx), lambda i, j: (i, j, 0)),
        pl.BlockSpec((1, 16, fx), lambda i, j: (i, 0, 0)),
        pl.BlockSpec((1, 1, 1, nb), lambda i, j: (i, j, 0, 0)),
        pl.BlockSpec((1, NC, nb, HW), lambda i, j: (i, 0, j, 0)),
    ] + [full(w.shape) for w in weights]
    return pl.pallas_call(
        _node_body,
        grid=grid,
        in_specs=in_specs,
        out_specs=pl.BlockSpec((1, 16, 64), lambda i, j: (i, 0, 0)),
        out_shape=jax.ShapeDtypeStruct((g, 16, 64), jnp.float32),
    )(x_s, u_s, batch_r, s_s, *weights)


def _final_body(u_ref, w0_ref, b0_ref, w1_ref, b1_ref, out_ref):
    dot = functools.partial(jnp.dot, preferred_element_type=jnp.float32)
    uh = jnp.concatenate([u_ref[0], u_ref[1]], axis=1)
    h = _relu(dot(uh, w0_ref[...]) + b0_ref[...])
    out_ref[...] = dot(h, w1_ref[...]) + b1_ref[...]


def _final(u_h, w0, b0, w1, b1):
    return pl.pallas_call(
        _final_body,
        out_shape=jax.ShapeDtypeStruct((16, w1.shape[1]), jnp.float32),
    )(u_h, w0, b0, w1, b1)


def kernel(x1, edge_index1, e1, u1, batch1, x2, edge_index2, e2, u2, batch2,
           params):
    n = x1.shape[0]
    n_edges = e1.shape[0]
    fx = x1.shape[1]
    fe = e1.shape[1]
    nb = 2000
    eb = 8000

    (w0, b0), (w1, b1) = params['edge']
    (wn0, bn0), (wn1, bn1) = params['node']
    (wa0, ba0), (wa1, ba1) = params['glob_a']
    (wv0, bv0), (wv1, bv1) = params['glob_v']
    (wf0, bf0), (wf1, bf1) = params['final']

    w0x = w0[:fx]
    w0e = w0[fx:fx + fe]
    w0u = w0[fx + fe:]
    wn0x = wn0[:fx]
    wn0a = wn0[fx:fx + 64]
    wn0u = wn0[fx + 64:]
    wa0x, wa0u = wa0[:64], wa0[64:]
    wv0x, wv0u = wv0[:64], wv0[64:]
    row = lambda v: v.reshape(1, -1)

    x_s = jnp.stack([x1, x2])
    e_s = jnp.stack([e1, e2])
    u_s = jnp.stack([u1, u2])
    batch_r = jnp.stack([batch1, batch2]).reshape(2, n // nb, 1, nb)

    ewb = _ewb(e_s, w0e, row(b0), eb)
    pq_s = _pq(x_s, u_s, batch_r, w0x, w0u, nb)

    n_pad = (-(-(n // NS) // 8) * 8) * NS
    zeros = jnp.zeros((n_pad, HW), jnp.float32)
    s1 = _edge_scatter(pq_s[0], ewb[0],
                       edge_index1[0], edge_index1[1], zeros)
    s2 = _edge_scatter(pq_s[1], ewb[1],
                       edge_index2[0], edge_index2[1], zeros)
    s_s = jnp.stack([s1, s2])

    weights = [w1, wn0x, wn0a, wn0u, row(bn0), wn1, row(bn1),
               wa0x, wa0u, row(ba0), wa1, row(ba1),
               wv0x, wv0u, row(bv0), wv1, row(bv1), row(b1)]
    u_h = _node_stage(x_s, u_s, batch_r, s_s, weights, nb)

    return _final(u_h, wf0, row(bf0), wf1, row(bf1))


# speedup vs baseline: 8.0699x; 1.3500x over previous
import functools

import jax
import jax.numpy as jnp
from jax import lax
from jax.experimental import pallas as pl
from jax.experimental.pallas import tpu as pltpu
import jax.experimental.pallas.tpu_sc as plsc

NC = 2
NS = 16
CHUNK = 40
HW = 128


def _relu(v):
    return jnp.maximum(v, 0.0)


def _sigmoid(v):
    return 1.0 / (1.0 + jnp.exp(-v))


def _ewb_body(e_ref, w_ref, b_ref, out_ref):
    out_ref[...] = jnp.dot(e_ref[...], w_ref[...],
                           preferred_element_type=jnp.float32) + b_ref[...]


def _ewb(e, w0e, b0, eb):
    e_tot, fe = e.shape
    return pl.pallas_call(
        _ewb_body,
        grid=(e_tot // eb,),
        in_specs=[
            pl.BlockSpec((eb, fe), lambda j: (j, 0)),
            pl.BlockSpec((fe, 64), lambda j: (0, 0)),
            pl.BlockSpec((1, 64), lambda j: (0, 0)),
        ],
        out_specs=pl.BlockSpec((eb, 64), lambda j: (j, 0)),
        out_shape=jax.ShapeDtypeStruct((e_tot, 64), jnp.float32),
    )(e, w0e, b0)


def _pq_body(x_ref, u_ref, b_ref, wx_ref, wu_ref, pq_ref):
    nb = x_ref.shape[0]
    xw = jnp.dot(x_ref[...], wx_ref[...], preferred_element_type=jnp.float32)
    uw = jnp.dot(u_ref[...], wu_ref[...], preferred_element_type=jnp.float32)
    bvec = b_ref[0, 0]
    oh = (bvec.reshape(nb, 1) ==
          lax.broadcasted_iota(jnp.int32, (nb, 16), 1)).astype(jnp.float32)
    ug = jnp.dot(oh, uw, preferred_element_type=jnp.float32)
    pq_ref[...] = jnp.concatenate([xw, ug - xw], axis=1)


def _pq(x, u, batch_r, w0x, w0u, nb):
    n_tot, fx = x.shape
    return pl.pallas_call(
        _pq_body,
        grid=(n_tot // nb,),
        in_specs=[
            pl.BlockSpec((nb, fx), lambda j: (j, 0)),
            pl.BlockSpec((16, fx), lambda j: (0, 0)),
            pl.BlockSpec((1, 1, nb), lambda j: (j, 0, 0)),
            pl.BlockSpec((fx, 64), lambda j: (0, 0)),
            pl.BlockSpec((fx, 64), lambda j: (0, 0)),
        ],
        out_specs=pl.BlockSpec((nb, 128), lambda j: (j, 0)),
        out_shape=jax.ShapeDtypeStruct((n_tot, 128), jnp.float32),
    )(x, u, batch_r, w0x, w0u)


def _edge_sc_body(n_pad, n_edges,
                  pq_hbm, ewb_hbm, src_hbm, dst_hbm, z_hbm, out_hbm,
                  idx_s, idx_d, buf_d, buf_s, buf_e, buf_h, s_sh,
                  sem_p0, sem_q0, sem_e0, sem_p1, sem_q1, sem_e1):
    c = lax.axis_index("c")
    s = lax.axis_index("s")
    w = s * NC + c
    npart = n_pad // NS
    per_tile = n_edges // (NC * NS)
    nchunk = per_tile // CHUNK
    base0 = w * per_tile
    sems = ((sem_p0, sem_q0, sem_e0), (sem_p1, sem_q1, sem_e1))

    pltpu.sync_copy(z_hbm.at[pl.ds(s * npart, npart)],
                    s_sh.at[pl.ds(s * npart, npart)])

    def ones_row(r, carry):
        buf_h[r, pl.ds(64, 16)] = jnp.full((16,), 1.0, jnp.float32)
        for k in range(5, 8):
            buf_h[r, pl.ds(k * 16, 16)] = jnp.zeros((16,), jnp.float32)
        return carry
    lax.fori_loop(0, CHUNK, ones_row, 0)
    plsc.subcore_barrier()

    def fire(b, base):
        sp, sq, se = sems[b]
        pltpu.sync_copy(src_hbm.at[pl.ds(base, CHUNK)], idx_s.at[b])
        pltpu.sync_copy(dst_hbm.at[pl.ds(base, CHUNK)], idx_d.at[b])
        pltpu.async_copy(pq_hbm.at[idx_d.at[b]], buf_d.at[b], sp)
        pltpu.async_copy(pq_hbm.at[idx_s.at[b]], buf_s.at[b], sq)
        pltpu.async_copy(ewb_hbm.at[pl.ds(base, CHUNK)], buf_e.at[b], se)

    def consume(b, base):
        sp, sq, se = sems[b]
        pltpu.make_async_copy(pq_hbm.at[idx_d.at[b]], buf_d.at[b], sp).wait()
        pltpu.make_async_copy(pq_hbm.at[idx_s.at[b]], buf_s.at[b], sq).wait()
        pltpu.make_async_copy(ewb_hbm.at[pl.ds(base, CHUNK)], buf_e.at[b],
                              se).wait()
        bd = buf_d.at[b]
        bs = buf_s.at[b]
        be = buf_e.at[b]

        def row(r, inner):
            for k in range(4):
                sl = pl.ds(k * 16, 16)
                buf_h[r, sl] = _relu(bd[r, sl]
                                     + bs[r, pl.ds(64 + k * 16, 16)]
                                     + be[r, sl])
            return inner
        lax.fori_loop(0, CHUNK, row, 0)
        pltpu.sync_copy(buf_h, s_sh.at[idx_d.at[b]], add=True)

    fire(0, base0)
    fire(1, base0 + CHUNK)

    def pair_body(j2, carry):
        j = j2 * 2
        for b in range(2):
            base = base0 + (j + b) * CHUNK
            consume(b, base)
            fire(b, base + 2 * CHUNK)
        return carry
    lax.fori_loop(0, (nchunk - 2) // 2, pair_body, 0)

    consume(0, base0 + (nchunk - 2) * CHUNK)
    consume(1, base0 + (nchunk - 1) * CHUNK)

    plsc.subcore_barrier()
    pltpu.sync_copy(s_sh.at[pl.ds(s * npart, npart)],
                    out_hbm.at[c, pl.ds(s * npart, npart)])


def _edge_scatter(pq, ewb, src, dst, zeros):
    n_pad = zeros.shape[0]
    n_edges = src.shape[0]
    mesh = plsc.VectorSubcoreMesh(core_axis_name="c", subcore_axis_name="s",
                                  num_cores=NC, num_subcores=NS)
    kern = pl.kernel(
        functools.partial(_edge_sc_body, n_pad, n_edges),
        out_type=jax.ShapeDtypeStruct((NC, n_pad, HW), jnp.float32),
        mesh=mesh,
        scratch_types=[
            pltpu.VMEM((2, CHUNK), jnp.int32),
            pltpu.VMEM((2, CHUNK), jnp.int32),
            pltpu.VMEM((2, CHUNK, 128), jnp.float32),
            pltpu.VMEM((2, CHUNK, 128), jnp.float32),
            pltpu.VMEM((2, CHUNK, 64), jnp.float32),
            pltpu.VMEM((CHUNK, HW), jnp.float32),
            pltpu.VMEM_SHARED((n_pad, HW), jnp.float32),
        ] + [pltpu.SemaphoreType.DMA] * 6,
    )
    return kern(pq, ewb, src, dst, zeros)


def _node_body(x_ref, u_ref, b_ref, s_ref,
               w1e_ref, wn0x_ref, wn0a_ref, wn0u_ref, bn0_ref,
               wn1_ref, bn1_ref,
               wa0x_ref, wa0u_ref, ba0_ref, wa1_ref, ba1_ref,
               wv0x_ref, wv0u_ref, bv0_ref, wv1_ref, bv1_ref,
               b1_ref, out_ref):
    i = pl.program_id(0)
    nb = x_ref.shape[0]
    dot = functools.partial(jnp.dot, preferred_element_type=jnp.float32)

    ssum = s_ref[0] + s_ref[1]
    s64 = ssum[:, :64]
    deg = ssum[:, 64:65]
    x = x_ref[...]
    uu = u_ref[...]
    bvec = b_ref[0, 0]
    oh = (bvec.reshape(nb, 1) ==
          lax.broadcasted_iota(jnp.int32, (nb, 16), 1)).astype(jnp.float32)

    wf = dot(w1e_ref[...], wn0a_ref[...])
    bf = dot(b1_ref[...], wn0a_ref[...])

    z = _relu(dot(x, wn0x_ref[...]) + dot(s64, wf) + deg * bf
              + dot(oh, dot(uu, wn0u_ref[...])) + bn0_ref[...])
    x_h = dot(z, wn1_ref[...]) + bn1_ref[...]

    ga = _relu(dot(x_h, wa0x_ref[...]) + dot(oh, dot(uu, wa0u_ref[...]))
               + ba0_ref[...])
    attn = _sigmoid(dot(ga, wa1_ref[...]) + ba1_ref[...])
    gv = _relu(dot(x_h, wv0x_ref[...]) + dot(oh, dot(uu, wv0u_ref[...]))
               + bv0_ref[...])
    val = dot(gv, wv1_ref[...]) + bv1_ref[...]
    y = attn * val

    part = lax.dot_general(oh, y, (((0,), (0,)), ((), ())),
                           preferred_element_type=jnp.float32)

    @pl.when(i == 0)
    def _():
        out_ref[...] = part

    @pl.when(i > 0)
    def _():
        out_ref[...] += part


def _node_stage(x, u, batch_r, s, weights, nb):
    n_tot, fx = x.shape
    full = lambda shp: pl.BlockSpec(shp, lambda j: tuple(0 for _ in shp))
    in_specs = [
        pl.BlockSpec((nb, fx), lambda j: (j, 0)),
        pl.BlockSpec((16, fx), lambda j: (0, 0)),
        pl.BlockSpec((1, 1, nb), lambda j: (j, 0, 0)),
        pl.BlockSpec((NC, nb, HW), lambda j: (0, j, 0)),
    ] + [full(w.shape) for w in weights]
    return pl.pallas_call(
        _node_body,
        grid=(n_tot // nb,),
        in_specs=in_specs,
        out_specs=pl.BlockSpec((16, 64), lambda j: (0, 0)),
        out_shape=jax.ShapeDtypeStruct((16, 64), jnp.float32),
    )(x, u, batch_r, s, *weights)


def _final_body(u1_ref, u2_ref, w0_ref, b0_ref, w1_ref, b1_ref, out_ref):
    dot = functools.partial(jnp.dot, preferred_element_type=jnp.float32)
    uh = jnp.concatenate([u1_ref[...], u2_ref[...]], axis=1)
    h = _relu(dot(uh, w0_ref[...]) + b0_ref[...])
    out_ref[...] = dot(h, w1_ref[...]) + b1_ref[...]


def _final(u1_h, u2_h, w0, b0, w1, b1):
    return pl.pallas_call(
        _final_body,
        out_shape=jax.ShapeDtypeStruct((16, w1.shape[1]), jnp.float32),
    )(u1_h, u2_h, w0, b0, w1, b1)


def kernel(x1, edge_index1, e1, u1, batch1, x2, edge_index2, e2, u2, batch2,
           params):
    n = x1.shape[0]
    n_edges = e1.shape[0]
    fx = x1.shape[1]
    fe = e1.shape[1]
    nb = 2000
    eb = 8000

    (w0, b0), (w1, b1) = params['edge']
    (wn0, bn0), (wn1, bn1) = params['node']
    (wa0, ba0), (wa1, ba1) = params['glob_a']
    (wv0, bv0), (wv1, bv1) = params['glob_v']
    (wf0, bf0), (wf1, bf1) = params['final']

    w0x = w0[:fx]
    w0e = w0[fx:fx + fe]
    w0u = w0[fx + fe:]
    wn0x = wn0[:fx]
    wn0a = wn0[fx:fx + 64]
    wn0u = wn0[fx + 64:]
    wa0x, wa0u = wa0[:64], wa0[64:]
    wv0x, wv0u = wv0[:64], wv0[64:]
    row = lambda v: v.reshape(1, -1)

    batch_r1 = batch1.reshape(n // nb, 1, nb)
    batch_r2 = batch2.reshape(n // nb, 1, nb)

    n_pad = (-(-(n // NS) // 8) * 8) * NS
    zeros = jnp.zeros((n_pad, HW), jnp.float32)
    weights = [w1, wn0x, wn0a, wn0u, row(bn0), wn1, row(bn1),
               wa0x, wa0u, row(ba0), wa1, row(ba1),
               wv0x, wv0u, row(bv0), wv1, row(bv1), row(b1)]

    ewb1 = _ewb(e1, w0e, row(b0), eb)
    pq1 = _pq(x1, u1, batch_r1, w0x, w0u, nb)
    s1 = _edge_scatter(pq1, ewb1, edge_index1[0], edge_index1[1], zeros)

    ewb2 = _ewb(e2, w0e, row(b0), eb)
    pq2 = _pq(x2, u2, batch_r2, w0x, w0u, nb)
    s2 = _edge_scatter(pq2, ewb2, edge_index2[0], edge_index2[1], zeros)

    u1_h = _node_stage(x1, u1, batch_r1, s1, weights, nb)
    u2_h = _node_stage(x2, u2, batch_r2, s2, weights, nb)

    return _final(u1_h, u2_h, wf0, row(bf0), wf1, row(bf1))
